# baseline (speedup 1.0000x reference)
import functools

import jax
import jax.numpy as jnp
from jax import lax
from jax.experimental import pallas as pl
from jax.experimental.pallas import tpu as pltpu, tpu_sc as plsc

N_NODES = 10000
D_IN = 128
HIDDEN = 32
D_OUT = 16
N_EDGES = 640000

NC = 2
NS = 16
NW = NC * NS
CHUNK = 128
K_CHUNKS = 160
E_PAD = NW * K_CHUNKS * CHUNK
NBUF = 4
ACC_N = 10240
DUMP_ROW = N_NODES
STRIPE = ACC_N // NS


def _make_segsum(feat_dim):
    mesh = plsc.VectorSubcoreMesh(core_axis_name="c", subcore_axis_name="s")

    @functools.partial(
        pl.kernel,
        out_type=jax.ShapeDtypeStruct((NC, ACC_N, feat_dim), jnp.float32),
        mesh=mesh,
        scratch_types=[
            pltpu.VMEM((K_CHUNKS, CHUNK), jnp.int32),
            pltpu.VMEM((K_CHUNKS, CHUNK), jnp.int32),
            [pltpu.VMEM((CHUNK, feat_dim), jnp.float32)] * NBUF,
            pltpu.VMEM_SHARED((ACC_N, feat_dim), jnp.float32),
            [pltpu.SemaphoreType.DMA] * NBUF,
            [pltpu.SemaphoreType.DMA] * NBUF,
        ],
        compiler_params=pltpu.CompilerParams(use_tc_tiling_on_sc=False),
    )
    def segsum(h_hbm, src_hbm, dst_hbm, zero_hbm, out_hbm,
               srcv, dstv, rows, acc, gsem, ssem):
        c = lax.axis_index("c")
        s = lax.axis_index("s")
        wid = s * NC + c

        pltpu.sync_copy(zero_hbm.at[pl.ds(s * STRIPE, STRIPE)],
                        acc.at[pl.ds(s * STRIPE, STRIPE)])
        pltpu.sync_copy(src_hbm.at[wid], srcv)
        pltpu.sync_copy(dst_hbm.at[wid], dstv)
        plsc.subcore_barrier()

        for b in range(NBUF):
            pltpu.async_copy(h_hbm.at[srcv.at[b]], rows[b], gsem[b])

        @pl.loop(0, K_CHUNKS, step=NBUF)
        def _(j):
            for b in range(NBUF):
                pltpu.make_async_copy(
                    h_hbm.at[srcv.at[j + b]], rows[b], gsem[b]).wait()
                pltpu.async_copy(
                    rows[b], acc.at[dstv.at[j + b]], ssem[b], add=True)
            for b in range(NBUF):
                pltpu.make_async_copy(
                    rows[b], acc.at[dstv.at[j + b]], ssem[b]).wait()

                @pl.when(j + NBUF + b < K_CHUNKS)
                def _(b=b):
                    pltpu.async_copy(
                        h_hbm.at[srcv.at[j + NBUF + b]], rows[b], gsem[b])

        plsc.subcore_barrier()
        pltpu.sync_copy(acc.at[pl.ds(s * STRIPE, STRIPE)],
                        out_hbm.at[c, pl.ds(s * STRIPE, STRIPE)])

    return segsum


_segsum_h = _make_segsum(HIDDEN)
_segsum_z = _make_segsum(D_OUT)


def _mm0_body(x_ref, w_ref, o_ref):
    o_ref[...] = jnp.dot(x_ref[...], w_ref[...],
                         preferred_element_type=jnp.float32)


def _mm1_body(ha_ref, hb_ref, w_ref, o_ref):
    h = jnp.maximum(ha_ref[...] + hb_ref[...], 0.0)
    o_ref[...] = jnp.dot(h, w_ref[...], preferred_element_type=jnp.float32)


_PRED_BM = 400


def _pred_body(za_ref, zb_ref, z_ref, pred_ref):
    i = pl.program_id(0)
    zfull = za_ref[...] + zb_ref[...]
    zblk = za_ref[pl.ds(i * _PRED_BM, _PRED_BM), :] + \
        zb_ref[pl.ds(i * _PRED_BM, _PRED_BM), :]
    z_ref[...] = zblk
    pred_ref[...] = lax.dot_general(
        zblk, zfull, (((1,), (1,)), ((), ())),
        preferred_element_type=jnp.float32)


def kernel(x, edge_index, W0, W1):
    src = edge_index[0].astype(jnp.int32)
    dst = edge_index[1].astype(jnp.int32)
    pad = E_PAD - N_EDGES
    src_p = jnp.concatenate([src, jnp.zeros((pad,), jnp.int32)]
                            ).reshape(NW, K_CHUNKS, CHUNK)
    dst_p = jnp.concatenate([dst, jnp.full((pad,), DUMP_ROW, jnp.int32)]
                            ).reshape(NW, K_CHUNKS, CHUNK)
    zero_h = jnp.zeros((ACC_N, HIDDEN), jnp.float32)
    zero_z = jnp.zeros((ACC_N, D_OUT), jnp.float32)

    h0 = pl.pallas_call(
        _mm0_body,
        out_shape=jax.ShapeDtypeStruct((N_NODES, HIDDEN), jnp.float32),
    )(x, W0)

    h_parts = _segsum_h(h0, src_p, dst_p, zero_h)

    z0 = pl.pallas_call(
        _mm1_body,
        grid=(1,),
        in_specs=[
            pl.BlockSpec((N_NODES, HIDDEN), lambda i: (0, 0)),
            pl.BlockSpec((N_NODES, HIDDEN), lambda i: (0, 0)),
            pl.BlockSpec((HIDDEN, D_OUT), lambda i: (0, 0)),
        ],
        out_specs=pl.BlockSpec((N_NODES, D_OUT), lambda i: (0, 0)),
        out_shape=jax.ShapeDtypeStruct((N_NODES, D_OUT), jnp.float32),
    )(h_parts[0], h_parts[1], W1)

    z_parts = _segsum_z(z0, src_p, dst_p, zero_z)

    grid = N_NODES // _PRED_BM
    z, pred = pl.pallas_call(
        _pred_body,
        grid=(grid,),
        in_specs=[
            pl.BlockSpec((N_NODES, D_OUT), lambda i: (0, 0)),
            pl.BlockSpec((N_NODES, D_OUT), lambda i: (0, 0)),
        ],
        out_specs=[
            pl.BlockSpec((_PRED_BM, D_OUT), lambda i: (i, 0)),
            pl.BlockSpec((_PRED_BM, N_NODES), lambda i: (i, 0)),
        ],
        out_shape=[
            jax.ShapeDtypeStruct((N_NODES, D_OUT), jnp.float32),
            jax.ShapeDtypeStruct((N_NODES, N_NODES), jnp.float32),
        ],
    )(z_parts[0], z_parts[1])

    return z, pred.reshape(-1)


# speedup vs baseline: 1.0163x; 1.0163x over previous
import functools

import jax
import jax.numpy as jnp
from jax import lax
from jax.experimental import pallas as pl
from jax.experimental.pallas import tpu as pltpu, tpu_sc as plsc

N_NODES = 10000
D_IN = 128
HIDDEN = 32
D_OUT = 16
N_EDGES = 640000

NC = 2
NS = 16
NW = NC * NS
CHUNK = 128
K_CHUNKS = 160
E_PAD = NW * K_CHUNKS * CHUNK
NBUF = 4
ACC_N = 10240
DUMP_ROW = N_NODES
STRIPE = ACC_N // NS


def _make_segsum(feat_dim):
    mesh = plsc.VectorSubcoreMesh(core_axis_name="c", subcore_axis_name="s")

    @functools.partial(
        pl.kernel,
        out_type=jax.ShapeDtypeStruct((NC, ACC_N, feat_dim), jnp.float32),
        mesh=mesh,
        scratch_types=[
            pltpu.VMEM((K_CHUNKS, CHUNK), jnp.int32),
            pltpu.VMEM((K_CHUNKS, CHUNK), jnp.int32),
            [pltpu.VMEM((CHUNK, feat_dim), jnp.float32)] * NBUF,
            pltpu.VMEM_SHARED((ACC_N, feat_dim), jnp.float32),
            [pltpu.SemaphoreType.DMA] * NBUF,
            [pltpu.SemaphoreType.DMA] * NBUF,
        ],
        compiler_params=pltpu.CompilerParams(use_tc_tiling_on_sc=False),
    )
    def segsum(h_hbm, src_hbm, dst_hbm, zero_hbm, out_hbm,
               srcv, dstv, rows, acc, gsem, ssem):
        c = lax.axis_index("c")
        s = lax.axis_index("s")
        wid = s * NC + c

        pltpu.sync_copy(zero_hbm.at[pl.ds(s * STRIPE, STRIPE)],
                        acc.at[pl.ds(s * STRIPE, STRIPE)])
        pltpu.sync_copy(src_hbm.at[wid], srcv)
        pltpu.sync_copy(dst_hbm.at[wid], dstv)
        plsc.subcore_barrier()

        def issue_g(m, slot):
            pltpu.async_copy(h_hbm.at[srcv.at[m]], rows[slot], gsem[slot])

        def wait_g(m, slot):
            pltpu.make_async_copy(
                h_hbm.at[srcv.at[m]], rows[slot], gsem[slot]).wait()

        def issue_s(m, slot):
            pltpu.async_copy(
                rows[slot], acc.at[dstv.at[m]], ssem[slot], add=True)

        def wait_s(m, slot):
            pltpu.make_async_copy(
                rows[slot], acc.at[dstv.at[m]], ssem[slot]).wait()

        issue_g(0, 0)
        issue_g(1, 1)
        wait_g(0, 0)
        issue_s(0, 0)
        issue_g(2, 2)
        wait_g(1, 1)
        issue_s(1, 1)
        issue_g(3, 3)

        @pl.loop(2, K_CHUNKS - 2, step=NBUF)
        def _(j):
            for b in range(NBUF):
                s_now = (2 + b) % NBUF
                s_nxt = b
                wait_g(j + b, s_now)
                issue_s(j + b, s_now)
                wait_s(j + b - 2, s_nxt)
                issue_g(j + b + 2, s_nxt)

        m0 = K_CHUNKS - 2
        wait_g(m0, m0 % NBUF)
        issue_s(m0, m0 % NBUF)
        wait_s(m0 - 2, (m0 - 2) % NBUF)
        wait_g(m0 + 1, (m0 + 1) % NBUF)
        issue_s(m0 + 1, (m0 + 1) % NBUF)
        wait_s(m0 - 1, (m0 - 1) % NBUF)
        wait_s(m0, m0 % NBUF)
        wait_s(m0 + 1, (m0 + 1) % NBUF)

        plsc.subcore_barrier()
        pltpu.sync_copy(acc.at[pl.ds(s * STRIPE, STRIPE)],
                        out_hbm.at[c, pl.ds(s * STRIPE, STRIPE)])

    return segsum


_segsum_h = _make_segsum(HIDDEN)
_segsum_z = _make_segsum(D_OUT)


def _mm0_body(x_ref, w_ref, o_ref):
    o_ref[...] = jnp.dot(x_ref[...], w_ref[...],
                         preferred_element_type=jnp.float32)


def _mm1_body(ha_ref, hb_ref, w_ref, o_ref):
    h = jnp.maximum(ha_ref[...] + hb_ref[...], 0.0)
    o_ref[...] = jnp.dot(h, w_ref[...], preferred_element_type=jnp.float32)


_PRED_BM = 400


def _pred_body(za_ref, zb_ref, z_ref, pred_ref):
    i = pl.program_id(0)
    zfull = za_ref[...] + zb_ref[...]
    zblk = za_ref[pl.ds(i * _PRED_BM, _PRED_BM), :] + \
        zb_ref[pl.ds(i * _PRED_BM, _PRED_BM), :]
    z_ref[...] = zblk
    pred_ref[...] = lax.dot_general(
        zblk, zfull, (((1,), (1,)), ((), ())),
        preferred_element_type=jnp.float32)


def kernel(x, edge_index, W0, W1):
    src = edge_index[0].astype(jnp.int32)
    dst = edge_index[1].astype(jnp.int32)
    pad = E_PAD - N_EDGES
    src_p = jnp.concatenate([src, jnp.zeros((pad,), jnp.int32)]
                            ).reshape(NW, K_CHUNKS, CHUNK)
    dst_p = jnp.concatenate([dst, jnp.full((pad,), DUMP_ROW, jnp.int32)]
                            ).reshape(NW, K_CHUNKS, CHUNK)
    zero_h = jnp.zeros((ACC_N, HIDDEN), jnp.float32)
    zero_z = jnp.zeros((ACC_N, D_OUT), jnp.float32)

    h0 = pl.pallas_call(
        _mm0_body,
        out_shape=jax.ShapeDtypeStruct((N_NODES, HIDDEN), jnp.float32),
    )(x, W0)

    h_parts = _segsum_h(h0, src_p, dst_p, zero_h)

    z0 = pl.pallas_call(
        _mm1_body,
        grid=(1,),
        in_specs=[
            pl.BlockSpec((N_NODES, HIDDEN), lambda i: (0, 0)),
            pl.BlockSpec((N_NODES, HIDDEN), lambda i: (0, 0)),
            pl.BlockSpec((HIDDEN, D_OUT), lambda i: (0, 0)),
        ],
        out_specs=pl.BlockSpec((N_NODES, D_OUT), lambda i: (0, 0)),
        out_shape=jax.ShapeDtypeStruct((N_NODES, D_OUT), jnp.float32),
    )(h_parts[0], h_parts[1], W1)

    z_parts = _segsum_z(z0, src_p, dst_p, zero_z)

    grid = N_NODES // _PRED_BM
    z, pred = pl.pallas_call(
        _pred_body,
        grid=(grid,),
        in_specs=[
            pl.BlockSpec((N_NODES, D_OUT), lambda i: (0, 0)),
            pl.BlockSpec((N_NODES, D_OUT), lambda i: (0, 0)),
        ],
        out_specs=[
            pl.BlockSpec((_PRED_BM, D_OUT), lambda i: (i, 0)),
            pl.BlockSpec((_PRED_BM, N_NODES), lambda i: (i, 0)),
        ],
        out_shape=[
            jax.ShapeDtypeStruct((N_NODES, D_OUT), jnp.float32),
            jax.ShapeDtypeStruct((N_NODES, N_NODES), jnp.float32),
        ],
    )(z_parts[0], z_parts[1])

    return z, pred.reshape(-1)


# speedup vs baseline: 1.6488x; 1.6224x over previous
import functools

import jax
import jax.numpy as jnp
from jax import lax
from jax.experimental import pallas as pl
from jax.experimental.pallas import tpu as pltpu, tpu_sc as plsc

N_NODES = 10000
D_IN = 128
HIDDEN = 32
D_OUT = 16
N_EDGES = 640000

NC = 2
NS = 16
NW = NC * NS
CHUNK = 128
K_CHUNKS = 158
E_PAD = NW * K_CHUNKS * CHUNK
ACC_N = 10240
DUMP_ROW = N_NODES
STRIPE = ACC_N // NS


def _make_segsum(feat_dim):
    mesh = plsc.VectorSubcoreMesh(core_axis_name="c", subcore_axis_name="s")

    @functools.partial(
        pl.kernel,
        out_type=jax.ShapeDtypeStruct((NC, ACC_N, feat_dim), jnp.float32),
        mesh=mesh,
        scratch_types=[
            pltpu.VMEM((K_CHUNKS, CHUNK), jnp.int32),
            pltpu.VMEM((K_CHUNKS, CHUNK), jnp.int32),
            pltpu.VMEM((CHUNK, feat_dim), jnp.float32),
            pltpu.VMEM((CHUNK, feat_dim), jnp.float32),
            pltpu.VMEM_SHARED((ACC_N, feat_dim), jnp.float32),
            pltpu.SemaphoreType.DMA,
            pltpu.SemaphoreType.DMA,
        ],
        compiler_params=pltpu.CompilerParams(use_tc_tiling_on_sc=False),
    )
    def segsum(h_hbm, src_hbm, dst_hbm, zero_hbm, out_hbm,
               srcv, dstv, rows0, rows1, acc, sem0, sem1):
        c = lax.axis_index("c")
        s = lax.axis_index("s")
        wid = s * NC + c

        pltpu.sync_copy(zero_hbm.at[pl.ds(s * STRIPE, STRIPE)],
                        acc.at[pl.ds(s * STRIPE, STRIPE)])
        pltpu.sync_copy(src_hbm.at[wid], srcv)
        pltpu.sync_copy(dst_hbm.at[wid], dstv)
        plsc.subcore_barrier()

        pltpu.async_copy(h_hbm.at[srcv.at[0]], rows0, sem0)

        @pl.loop(0, K_CHUNKS, step=2)
        def _(j):
            pltpu.async_copy(h_hbm.at[srcv.at[j + 1]], rows1, sem1)
            pltpu.make_async_copy(h_hbm.at[srcv.at[j]], rows0, sem0).wait()
            pltpu.sync_copy(rows0, acc.at[dstv.at[j]], add=True)

            @pl.when(j + 2 < K_CHUNKS)
            def _():
                pltpu.async_copy(h_hbm.at[srcv.at[j + 2]], rows0, sem0)

            pltpu.make_async_copy(h_hbm.at[srcv.at[j + 1]], rows1, sem1).wait()
            pltpu.sync_copy(rows1, acc.at[dstv.at[j + 1]], add=True)

        plsc.subcore_barrier()
        pltpu.sync_copy(acc.at[pl.ds(s * STRIPE, STRIPE)],
                        out_hbm.at[c, pl.ds(s * STRIPE, STRIPE)])

    return segsum


_segsum_h = _make_segsum(HIDDEN)
_segsum_z = _make_segsum(D_OUT)


def _mm0_body(x_ref, w_ref, o_ref):
    o_ref[...] = jnp.dot(x_ref[...], w_ref[...],
                         preferred_element_type=jnp.float32)


def _mm1_body(ha_ref, hb_ref, w_ref, o_ref):
    h = jnp.maximum(ha_ref[...] + hb_ref[...], 0.0)
    o_ref[...] = jnp.dot(h, w_ref[...], preferred_element_type=jnp.float32)


_PRED_BM = 64


def _pred_body(za_ref, zb_ref, z_ref, pred_ref, mat_ref):
    i = pl.program_id(0)
    zfull = za_ref[pl.ds(0, N_NODES), :] + zb_ref[pl.ds(0, N_NODES), :]
    zblk = za_ref[pl.ds(i * _PRED_BM, _PRED_BM), :] + \
        zb_ref[pl.ds(i * _PRED_BM, _PRED_BM), :]
    z_ref[...] = zblk
    mat_ref[...] = lax.dot_general(
        zblk, zfull, (((1,), (1,)), ((), ())),
        preferred_element_type=jnp.float32)
    for r in range(_PRED_BM):
        pred_ref[pl.ds(r * N_NODES, N_NODES)] = mat_ref[r, :]


def kernel(x, edge_index, W0, W1):
    src = edge_index[0].astype(jnp.int32)
    dst = edge_index[1].astype(jnp.int32)
    pad = E_PAD - N_EDGES
    src_p = jnp.concatenate([src, jnp.zeros((pad,), jnp.int32)]
                            ).reshape(NW, K_CHUNKS, CHUNK)
    dst_p = jnp.concatenate([dst, jnp.full((pad,), DUMP_ROW, jnp.int32)]
                            ).reshape(NW, K_CHUNKS, CHUNK)
    zero_h = jnp.zeros((ACC_N, HIDDEN), jnp.float32)
    zero_z = jnp.zeros((ACC_N, D_OUT), jnp.float32)

    h0 = pl.pallas_call(
        _mm0_body,
        out_shape=jax.ShapeDtypeStruct((N_NODES, HIDDEN), jnp.float32),
    )(x, W0)

    h_parts = _segsum_h(h0, src_p, dst_p, zero_h)

    z0 = pl.pallas_call(
        _mm1_body,
        grid=(1,),
        in_specs=[
            pl.BlockSpec((N_NODES, HIDDEN), lambda i: (0, 0)),
            pl.BlockSpec((N_NODES, HIDDEN), lambda i: (0, 0)),
            pl.BlockSpec((HIDDEN, D_OUT), lambda i: (0, 0)),
        ],
        out_specs=pl.BlockSpec((N_NODES, D_OUT), lambda i: (0, 0)),
        out_shape=jax.ShapeDtypeStruct((N_NODES, D_OUT), jnp.float32),
    )(h_parts[0], h_parts[1], W1)

    z_parts = _segsum_z(z0, src_p, dst_p, zero_z)

    grid = -(-N_NODES // _PRED_BM)
    z, pred = pl.pallas_call(
        _pred_body,
        grid=(grid,),
        in_specs=[
            pl.BlockSpec((ACC_N, D_OUT), lambda i: (0, 0)),
            pl.BlockSpec((ACC_N, D_OUT), lambda i: (0, 0)),
        ],
        out_specs=[
            pl.BlockSpec((_PRED_BM, D_OUT), lambda i: (i, 0)),
            pl.BlockSpec((_PRED_BM * N_NODES,), lambda i: (i,)),
        ],
        out_shape=[
            jax.ShapeDtypeStruct((N_NODES, D_OUT), jnp.float32),
            jax.ShapeDtypeStruct((N_NODES * N_NODES,), jnp.float32),
        ],
        scratch_shapes=[pltpu.VMEM((_PRED_BM, N_NODES), jnp.float32)],
    )(z_parts[0], z_parts[1])

    return z, pred


# speedup vs baseline: 1.8974x; 1.1508x over previous
import functools

import jax
import jax.numpy as jnp
from jax import lax
from jax.experimental import pallas as pl
from jax.experimental.pallas import tpu as pltpu, tpu_sc as plsc

N_NODES = 10000
D_IN = 128
HIDDEN = 32
D_OUT = 16
N_EDGES = 640000

NC = 2
NS = 16
NW = NC * NS
CHUNK = 128
K_CHUNKS = 158
E_PAD = NW * K_CHUNKS * CHUNK
ACC_N = 10240
DUMP_ROW = N_NODES
STRIPE = ACC_N // NS


def _make_segsum(feat_dim):
    mesh = plsc.VectorSubcoreMesh(core_axis_name="c", subcore_axis_name="s")

    @functools.partial(
        pl.kernel,
        out_type=jax.ShapeDtypeStruct((NC, ACC_N, feat_dim), jnp.float32),
        mesh=mesh,
        scratch_types=[
            pltpu.VMEM((K_CHUNKS, CHUNK), jnp.int32),
            pltpu.VMEM((K_CHUNKS, CHUNK), jnp.int32),
            pltpu.VMEM((CHUNK, feat_dim), jnp.float32),
            pltpu.VMEM((CHUNK, feat_dim), jnp.float32),
            pltpu.VMEM_SHARED((ACC_N, feat_dim), jnp.float32),
            pltpu.SemaphoreType.DMA,
            pltpu.SemaphoreType.DMA,
        ],
        compiler_params=pltpu.CompilerParams(use_tc_tiling_on_sc=False),
    )
    def segsum(h_hbm, src_hbm, dst_hbm, zero_hbm, out_hbm,
               srcv, dstv, rows0, rows1, acc, sem0, sem1):
        c = lax.axis_index("c")
        s = lax.axis_index("s")
        wid = s * NC + c

        pltpu.sync_copy(zero_hbm.at[pl.ds(s * STRIPE, STRIPE)],
                        acc.at[pl.ds(s * STRIPE, STRIPE)])
        pltpu.sync_copy(src_hbm.at[wid], srcv)
        pltpu.sync_copy(dst_hbm.at[wid], dstv)
        plsc.subcore_barrier()

        pltpu.async_copy(h_hbm.at[srcv.at[0]], rows0, sem0)

        @pl.loop(0, K_CHUNKS, step=2)
        def _(j):
            pltpu.async_copy(h_hbm.at[srcv.at[j + 1]], rows1, sem1)
            pltpu.make_async_copy(h_hbm.at[srcv.at[j]], rows0, sem0).wait()
            pltpu.sync_copy(rows0, acc.at[dstv.at[j]], add=True)

            @pl.when(j + 2 < K_CHUNKS)
            def _():
                pltpu.async_copy(h_hbm.at[srcv.at[j + 2]], rows0, sem0)

            pltpu.make_async_copy(h_hbm.at[srcv.at[j + 1]], rows1, sem1).wait()
            pltpu.sync_copy(rows1, acc.at[dstv.at[j + 1]], add=True)

        plsc.subcore_barrier()
        pltpu.sync_copy(acc.at[pl.ds(s * STRIPE, STRIPE)],
                        out_hbm.at[c, pl.ds(s * STRIPE, STRIPE)])

    return segsum


_segsum_h = _make_segsum(HIDDEN)
_segsum_z = _make_segsum(D_OUT)


def _mm0_body(x_ref, w_ref, o_ref):
    o_ref[...] = jnp.dot(x_ref[...], w_ref[...],
                         preferred_element_type=jnp.float32)


def _mm1_body(ha_ref, hb_ref, w_ref, o_ref):
    h = jnp.maximum(ha_ref[...] + hb_ref[...], 0.0)
    o_ref[...] = jnp.dot(h, w_ref[...], preferred_element_type=jnp.float32)


_PRED_BM = 256


def _pred_body(za_ref, zb_ref, z_ref, pred_ref, mat_ref):
    i = pl.program_id(0)
    zfull = za_ref[pl.ds(0, N_NODES), :] + zb_ref[pl.ds(0, N_NODES), :]
    zblk = za_ref[pl.ds(i * _PRED_BM, _PRED_BM), :] + \
        zb_ref[pl.ds(i * _PRED_BM, _PRED_BM), :]
    z_ref[...] = zblk
    mat_ref[...] = lax.dot_general(
        zblk, zfull, (((1,), (1,)), ((), ())),
        preferred_element_type=jnp.float32)
    for r in range(_PRED_BM):
        pred_ref[pl.ds(r * N_NODES, N_NODES)] = mat_ref[r, :]


def kernel(x, edge_index, W0, W1):
    src = edge_index[0].astype(jnp.int32)
    dst = edge_index[1].astype(jnp.int32)
    pad = E_PAD - N_EDGES
    src_p = jnp.concatenate([src, jnp.zeros((pad,), jnp.int32)]
                            ).reshape(NW, K_CHUNKS, CHUNK)
    dst_p = jnp.concatenate([dst, jnp.full((pad,), DUMP_ROW, jnp.int32)]
                            ).reshape(NW, K_CHUNKS, CHUNK)
    zero_h = jnp.zeros((ACC_N, HIDDEN), jnp.float32)
    zero_z = jnp.zeros((ACC_N, D_OUT), jnp.float32)

    h0 = pl.pallas_call(
        _mm0_body,
        out_shape=jax.ShapeDtypeStruct((N_NODES, HIDDEN), jnp.float32),
    )(x, W0)

    h_parts = _segsum_h(h0, src_p, dst_p, zero_h)

    z0 = pl.pallas_call(
        _mm1_body,
        grid=(1,),
        in_specs=[
            pl.BlockSpec((N_NODES, HIDDEN), lambda i: (0, 0)),
            pl.BlockSpec((N_NODES, HIDDEN), lambda i: (0, 0)),
            pl.BlockSpec((HIDDEN, D_OUT), lambda i: (0, 0)),
        ],
        out_specs=pl.BlockSpec((N_NODES, D_OUT), lambda i: (0, 0)),
        out_shape=jax.ShapeDtypeStruct((N_NODES, D_OUT), jnp.float32),
    )(h_parts[0], h_parts[1], W1)

    z_parts = _segsum_z(z0, src_p, dst_p, zero_z)

    grid = -(-N_NODES // _PRED_BM)
    z, pred = pl.pallas_call(
        _pred_body,
        grid=(grid,),
        in_specs=[
            pl.BlockSpec((ACC_N, D_OUT), lambda i: (0, 0)),
            pl.BlockSpec((ACC_N, D_OUT), lambda i: (0, 0)),
        ],
        out_specs=[
            pl.BlockSpec((_PRED_BM, D_OUT), lambda i: (i, 0)),
            pl.BlockSpec((_PRED_BM * N_NODES,), lambda i: (i,)),
        ],
        out_shape=[
            jax.ShapeDtypeStruct((N_NODES, D_OUT), jnp.float32),
            jax.ShapeDtypeStruct((N_NODES * N_NODES,), jnp.float32),
        ],
        scratch_shapes=[pltpu.VMEM((_PRED_BM, N_NODES), jnp.float32)],
    )(z_parts[0], z_parts[1])

    return z, pred
